# baseline (speedup 1.0000x reference)
import functools

import jax
import jax.numpy as jnp
from jax.experimental import pallas as pl
from jax.experimental.pallas import tpu as pltpu


def _rup(v, m):
    return (v + m - 1) // m * m


def _s2d(x, pad):
    B, H, W, C = x.shape
    Hp = H + 2 * pad
    P = Hp // 2
    xp = jnp.pad(x, ((0, 0), (pad, pad), (pad, pad), (0, 0)))
    v = xp.reshape(B, P, 2, P, 2, C).transpose(0, 1, 3, 2, 4, 5)
    return v.reshape(B, P * P, 4 * C), P


def _pack_w_s2(w_flat, kh, cin, cout):
    k2 = (kh + 1) // 2
    w = w_flat.reshape(kh, kh, cin, cout)
    w = jnp.pad(w, ((0, 2 * k2 - kh), (0, 2 * k2 - kh), (0, 0), (0, 0)))
    w = w.reshape(k2, 2, k2, 2, cin, cout).transpose(0, 2, 1, 3, 4, 5)
    return w.reshape(k2 * k2, 4 * cin, cout)


def _stem_kernel(p_ref, m_ref, s_ref, w_ref, b_ref, oe_ref, oo_ref,
                 xs_ref, yc_ref, *, P, Ho, TH):
    m = m_ref[:, 0:1]
    s = s_ref[:, 0:1]
    xn = ((p_ref[...] - m) * s).astype(jnp.bfloat16)
    t = jnp.transpose(xn, (1, 0))
    xs_ref[0:P * P, :] = t
    xs_ref[P * P:, :] = jnp.zeros((xs_ref.shape[0] - P * P, 12), jnp.bfloat16)
    ninf = jnp.finfo(jnp.float32).min
    Wh = (Ho + 2) // 2
    ce = jnp.full((TH, 1, 32), ninf, jnp.float32)
    re = jnp.full((1, Wh, 32), ninf, jnp.bfloat16)
    oe_ref[0:1] = re
    oe_ref[Ho + 1:Ho + 2] = re
    oo_ref[0:1] = re
    oo_ref[Ho + 1:Ho + 2] = re
    for A0 in range(0, Ho, TH):
        acc = None
        for t_i in range(16):
            off = A0 * P + (t_i // 4) * P + (t_i % 4)
            d = jnp.dot(xs_ref[off:off + TH * P, :], w_ref[t_i],
                        preferred_element_type=jnp.float32)
            acc = d if acc is None else acc + d
        y = jnp.maximum(acc + b_ref[...], 0.0)
        yc_ref[...] = y.reshape(TH, P, 32)
        ve = jnp.concatenate([ce, yc_ref[:, pl.ds(1, Wh - 1, 2), :]], axis=1)
        vo = jnp.concatenate([yc_ref[:, pl.ds(0, Wh - 1, 2), :], ce], axis=1)
        oe_ref[A0 + 1:A0 + TH + 1] = ve.astype(jnp.bfloat16)
        oo_ref[A0 + 1:A0 + TH + 1] = vo.astype(jnp.bfloat16)


def _stem(x_nchw, norm_mean, norm_std, stem_w, stem_shift):
    B = x_nchw.shape[0]
    B_, C_, H_, W_ = x_nchw.shape
    xp = jnp.broadcast_to(norm_mean.reshape(1, C_, 1, 1), (B_, C_, H_ + 6, W_ + 6))
    xp = xp.at[:, :, 3:H_ + 3, 3:W_ + 3].set(x_nchw)
    planes = jnp.stack([xp[:, c, pa::2, pb::2].reshape(B, 115 * 115)
                        for pa in (0, 1) for pb in (0, 1) for c in (0, 1, 2)],
                       axis=1)
    m12 = jnp.broadcast_to(jnp.tile(norm_mean, 4).reshape(12, 1), (12, 128))
    s12 = jnp.broadcast_to(jnp.tile(1.0 / norm_std, 4).reshape(12, 1), (12, 128))
    w16 = _pack_w_s2(stem_w, 7, 3, 32)
    P = 115
    Ho = 112
    Wh = 57
    TH = 56
    oe, oo = pl.pallas_call(
        functools.partial(_stem_kernel, P=P, Ho=Ho, TH=TH),
        out_shape=(jax.ShapeDtypeStruct((B, Ho + 2, Wh, 32), jnp.bfloat16),
                   jax.ShapeDtypeStruct((B, Ho + 2, Wh, 32), jnp.bfloat16)),
        grid=(B,),
        in_specs=[
            pl.BlockSpec((None, 12, P * P), lambda n: (n, 0, 0)),
            pl.BlockSpec((12, 128), lambda n: (0, 0)),
            pl.BlockSpec((12, 128), lambda n: (0, 0)),
            pl.BlockSpec((16, 12, 32), lambda n: (0, 0, 0)),
            pl.BlockSpec((1, 32), lambda n: (0, 0)),
        ],
        out_specs=(pl.BlockSpec((None, Ho + 2, Wh, 32), lambda n: (n, 0, 0, 0)),
                   pl.BlockSpec((None, Ho + 2, Wh, 32), lambda n: (n, 0, 0, 0))),
        scratch_shapes=[pltpu.VMEM((_rup(3 * P + 3 + Ho * P, 16), 12), jnp.bfloat16),
                        pltpu.VMEM((TH, P, 32), jnp.float32)],
        compiler_params=pltpu.CompilerParams(dimension_semantics=("parallel",)),
    )(planes, m12, s12, w16, stem_shift)
    return oe, oo


def _pool_kernel(xe_ref, xo_ref, o_ref, wm_ref, *, R, Wh):
    ninf = jnp.finfo(jnp.float32).min
    xe = xe_ref[0:R].astype(jnp.float32)
    xo = xo_ref[0:R].astype(jnp.float32)
    ce = jnp.full((R, 1, xe.shape[-1]), ninf, jnp.float32)
    sh = jnp.concatenate([xe[:, 1:Wh, :], ce], axis=1)
    wm_ref[...] = jnp.maximum(jnp.maximum(xe, xo), sh)
    t0 = wm_ref[pl.ds(0, (R - 1) // 2, 2)]
    t1 = wm_ref[pl.ds(1, (R - 1) // 2, 2)]
    t2 = wm_ref[pl.ds(2, (R - 1) // 2, 2)]
    o_ref[...] = jnp.maximum(jnp.maximum(t0, t1), t2).astype(o_ref.dtype)


def _maxpool(oe, oo):
    B, Hp, Wh, C = oe.shape
    Ho = (Hp - 2) // 2
    R = Hp - 1
    out = pl.pallas_call(
        functools.partial(_pool_kernel, R=R, Wh=Wh),
        out_shape=jax.ShapeDtypeStruct((B, Ho, Wh, C), jnp.bfloat16),
        grid=(B,),
        in_specs=[
            pl.BlockSpec((None, Hp, Wh, C), lambda n: (n, 0, 0, 0)),
            pl.BlockSpec((None, Hp, Wh, C), lambda n: (n, 0, 0, 0)),
        ],
        out_specs=pl.BlockSpec((None, Ho, Wh, C), lambda n: (n, 0, 0, 0)),
        scratch_shapes=[pltpu.VMEM((R, Wh, C), jnp.float32)],
        compiler_params=pltpu.CompilerParams(dimension_semantics=("parallel",)),
    )(oe, oo)
    return out[:, :, :Ho, :]


def _block_kernel(x_ref, wa_ref, ba_ref, wb_ref, sc_ref, sh_ref, o_ref, sp_ref,
                  *, P, Ho, W2):
    Qa = Ho * P
    acc = jnp.dot(x_ref[0:Qa, :], wa_ref[0], preferred_element_type=jnp.float32)
    for t in range(1, 4):
        off = (t // 2) * P + (t % 2)
        acc = acc + jnp.dot(x_ref[off:off + Qa, :], wa_ref[t],
                            preferred_element_type=jnp.float32)
    ya = jnp.maximum(acc + ba_ref[...], 0.0).astype(jnp.bfloat16)
    ya = ya.reshape(Ho, P, ya.shape[-1])
    sp_ref[...] = jnp.zeros_like(sp_ref)
    sp_ref[1:Ho + 1, 1:Ho + 1, :] = ya[:, 0:Ho, :]
    Qb = Ho * (Ho + 2)
    accb = None
    for ti in range(3):
        for tj in range(3):
            a = sp_ref[ti:ti + Ho, tj:tj + Ho + 2, :].reshape(Qb, sp_ref.shape[-1])
            d = jnp.dot(a, wb_ref[3 * ti + tj], preferred_element_type=jnp.float32)
            accb = d if accb is None else accb + d
    y = jnp.maximum(accb * sc_ref[...] + sh_ref[...], 0.0)
    o_ref[...] = y.astype(o_ref.dtype)


def _block(x, wa, shift_a, wb, scale_b, shift_b):
    B, Hi, _, Ci = x.shape
    Co = wb.shape[-1]
    Ho = Hi // 2
    xp, P = _s2d(x, 1)
    Qa = Ho * P
    Sp = _rup(P + 1 + Qa, 16)
    xp = jnp.pad(xp, ((0, 0), (0, Sp - P * P), (0, 0)))
    wa4 = _pack_w_s2(wa, 3, Ci, Co)
    W2 = Ho + 4
    Qb = Ho * (Ho + 2)
    out = pl.pallas_call(
        functools.partial(_block_kernel, P=P, Ho=Ho, W2=W2),
        out_shape=jax.ShapeDtypeStruct((B, Qb, Co), jnp.bfloat16),
        grid=(B,),
        in_specs=[
            pl.BlockSpec((None, Sp, 4 * Ci), lambda n: (n, 0, 0)),
            pl.BlockSpec((4, 4 * Ci, Co), lambda n: (0, 0, 0)),
            pl.BlockSpec((1, Co), lambda n: (0, 0)),
            pl.BlockSpec((9, Co, Co), lambda n: (0, 0, 0)),
            pl.BlockSpec((1, Co), lambda n: (0, 0)),
            pl.BlockSpec((1, Co), lambda n: (0, 0)),
        ],
        out_specs=pl.BlockSpec((None, Qb, Co), lambda n: (n, 0, 0)),
        scratch_shapes=[pltpu.VMEM((Ho + 2, W2, Co), jnp.bfloat16)],
        compiler_params=pltpu.CompilerParams(dimension_semantics=("parallel",)),
    )(xp, wa4, shift_a, wb, scale_b, shift_b)
    return out.reshape(B, Ho, Ho + 2, Co)[:, :, :Ho, :]


def _gap_kernel(x_ref, w_ref, b_ref, o_ref, *, inv_hw):
    z = jnp.sum(x_ref[...].astype(jnp.float32), axis=1) * inv_hw
    o_ref[...] = jnp.dot(z.astype(jnp.bfloat16), w_ref[...],
                         preferred_element_type=jnp.float32) + b_ref[...]


def _gap_linear(x, w, b):
    B, HW, C = x.shape
    ncls = w.shape[1]
    return pl.pallas_call(
        functools.partial(_gap_kernel, inv_hw=1.0 / HW),
        out_shape=jax.ShapeDtypeStruct((B, ncls), jnp.float32),
        grid=(1,),
        in_specs=[
            pl.BlockSpec((B, HW, C), lambda i: (0, 0, 0)),
            pl.BlockSpec((C, ncls), lambda i: (0, 0)),
            pl.BlockSpec((1, ncls), lambda i: (0, 0)),
        ],
        out_specs=pl.BlockSpec((B, ncls), lambda i: (0, 0)),
    )(x, w, b)


def kernel(x_nchw, norm_mean, norm_std, stem_w, stem_shift, fc_w, fc_b,
           b0_wa, b0_shift_a, b0_wb, b0_scale_b, b0_shift_b,
           b1_wa, b1_shift_a, b1_wb, b1_scale_b, b1_shift_b,
           b2_wa, b2_shift_a, b2_wb, b2_scale_b, b2_shift_b):
    oe, oo = _stem(x_nchw, norm_mean, norm_std, stem_w, stem_shift)
    x = _maxpool(oe, oo)
    for (wa, sa, wb, sc, sh) in (
            (b0_wa, b0_shift_a, b0_wb, b0_scale_b, b0_shift_b),
            (b1_wa, b1_shift_a, b1_wb, b1_scale_b, b1_shift_b),
            (b2_wa, b2_shift_a, b2_wb, b2_scale_b, b2_shift_b)):
        x = _block(x, wa, sa, wb, sc, sh)
    B, H, _, C = x.shape
    return _gap_linear(x.reshape(B, H * H, C), fc_w, fc_b)


# speedup vs baseline: 1.2527x; 1.2527x over previous
import functools

import jax
import jax.numpy as jnp
from jax.experimental import pallas as pl
from jax.experimental.pallas import tpu as pltpu


def _rup(v, m):
    return (v + m - 1) // m * m


def _s2d(x, pad):
    B, H, W, C = x.shape
    Hp = H + 2 * pad
    P = Hp // 2
    xp = jnp.pad(x, ((0, 0), (pad, pad), (pad, pad), (0, 0)))
    v = xp.reshape(B, P, 2, P, 2, C).transpose(0, 1, 3, 2, 4, 5)
    return v.reshape(B, P * P, 4 * C), P


def _pack_w_s2(w_flat, kh, cin, cout):
    k2 = (kh + 1) // 2
    w = w_flat.reshape(kh, kh, cin, cout)
    w = jnp.pad(w, ((0, 2 * k2 - kh), (0, 2 * k2 - kh), (0, 0), (0, 0)))
    w = w.reshape(k2, 2, k2, 2, cin, cout).transpose(0, 2, 1, 3, 4, 5)
    return w.reshape(k2 * k2, 4 * cin, cout)


def _stem_kernel(p_ref, m_ref, s_ref, w_ref, b_ref, oe_ref, oo_ref,
                 xs_ref, yc_ref, *, P, Ho, TH):
    m = m_ref[:, 0:1]
    s = s_ref[:, 0:1]
    xn = ((p_ref[...] - m) * s).astype(jnp.bfloat16)
    t = jnp.transpose(xn, (1, 0))
    xs_ref[0:P * P, :] = t
    xs_ref[P * P:, :] = jnp.zeros((xs_ref.shape[0] - P * P, 12), jnp.bfloat16)
    ninf = jnp.finfo(jnp.float32).min
    Wh = (Ho + 2) // 2
    ce = jnp.full((TH, 1, 32), ninf, jnp.float32)
    re = jnp.full((1, Wh, 32), ninf, jnp.bfloat16)
    oe_ref[0:1] = re
    oe_ref[Ho + 1:Ho + 2] = re
    oo_ref[0:1] = re
    oo_ref[Ho + 1:Ho + 2] = re
    for A0 in range(0, Ho, TH):
        acc = None
        for a in range(4):
            base = (A0 + a) * P
            v = jnp.concatenate(
                [xs_ref[base + b:base + b + TH * P, :] for b in range(4)],
                axis=1)
            d = jnp.dot(v, w_ref[a], preferred_element_type=jnp.float32)
            acc = d if acc is None else acc + d
        y = jnp.maximum(acc + b_ref[...], 0.0)
        yc_ref[...] = y.reshape(TH, P, 32)
        ve = jnp.concatenate([ce, yc_ref[:, pl.ds(1, Wh - 1, 2), :]], axis=1)
        vo = jnp.concatenate([yc_ref[:, pl.ds(0, Wh - 1, 2), :], ce], axis=1)
        oe_ref[A0 + 1:A0 + TH + 1] = ve.astype(jnp.bfloat16)
        oo_ref[A0 + 1:A0 + TH + 1] = vo.astype(jnp.bfloat16)


def _stem(x_nchw, norm_mean, norm_std, stem_w, stem_shift):
    B = x_nchw.shape[0]
    B_, C_, H_, W_ = x_nchw.shape
    xp = jnp.broadcast_to(norm_mean.reshape(1, C_, 1, 1), (B_, C_, H_ + 6, W_ + 6))
    xp = xp.at[:, :, 3:H_ + 3, 3:W_ + 3].set(x_nchw)
    planes = jnp.stack([xp[:, c, pa::2, pb::2].reshape(B, 115 * 115)
                        for pa in (0, 1) for pb in (0, 1) for c in (0, 1, 2)],
                       axis=1)
    m12 = jnp.broadcast_to(jnp.tile(norm_mean, 4).reshape(12, 1), (12, 128))
    s12 = jnp.broadcast_to(jnp.tile(1.0 / norm_std, 4).reshape(12, 1), (12, 128))
    w16 = _pack_w_s2(stem_w, 7, 3, 32)
    w48 = w16.reshape(4, 4 * 12, 32)
    P = 115
    Ho = 112
    Wh = 57
    TH = 56
    oe, oo = pl.pallas_call(
        functools.partial(_stem_kernel, P=P, Ho=Ho, TH=TH),
        out_shape=(jax.ShapeDtypeStruct((B, Ho + 2, Wh, 32), jnp.bfloat16),
                   jax.ShapeDtypeStruct((B, Ho + 2, Wh, 32), jnp.bfloat16)),
        grid=(B,),
        in_specs=[
            pl.BlockSpec((None, 12, P * P), lambda n: (n, 0, 0)),
            pl.BlockSpec((12, 128), lambda n: (0, 0)),
            pl.BlockSpec((12, 128), lambda n: (0, 0)),
            pl.BlockSpec((4, 48, 32), lambda n: (0, 0, 0)),
            pl.BlockSpec((1, 32), lambda n: (0, 0)),
        ],
        out_specs=(pl.BlockSpec((None, Ho + 2, Wh, 32), lambda n: (n, 0, 0, 0)),
                   pl.BlockSpec((None, Ho + 2, Wh, 32), lambda n: (n, 0, 0, 0))),
        scratch_shapes=[pltpu.VMEM((_rup(3 * P + 3 + Ho * P, 16), 12), jnp.bfloat16),
                        pltpu.VMEM((TH, P, 32), jnp.float32)],
        compiler_params=pltpu.CompilerParams(dimension_semantics=("parallel",)),
    )(planes, m12, s12, w48, stem_shift)
    return oe, oo


def _pool_kernel(xe_ref, xo_ref, o_ref, wm_ref, *, R, Wh):
    ninf = jnp.finfo(jnp.float32).min
    xe = xe_ref[0:R].astype(jnp.float32)
    xo = xo_ref[0:R].astype(jnp.float32)
    ce = jnp.full((R, 1, xe.shape[-1]), ninf, jnp.float32)
    sh = jnp.concatenate([xe[:, 1:Wh, :], ce], axis=1)
    wm_ref[...] = jnp.maximum(jnp.maximum(xe, xo), sh)
    t0 = wm_ref[pl.ds(0, (R - 1) // 2, 2)]
    t1 = wm_ref[pl.ds(1, (R - 1) // 2, 2)]
    t2 = wm_ref[pl.ds(2, (R - 1) // 2, 2)]
    o_ref[...] = jnp.maximum(jnp.maximum(t0, t1), t2).astype(o_ref.dtype)


def _maxpool(oe, oo):
    B, Hp, Wh, C = oe.shape
    Ho = (Hp - 2) // 2
    R = Hp - 1
    out = pl.pallas_call(
        functools.partial(_pool_kernel, R=R, Wh=Wh),
        out_shape=jax.ShapeDtypeStruct((B, Ho, Wh, C), jnp.bfloat16),
        grid=(B,),
        in_specs=[
            pl.BlockSpec((None, Hp, Wh, C), lambda n: (n, 0, 0, 0)),
            pl.BlockSpec((None, Hp, Wh, C), lambda n: (n, 0, 0, 0)),
        ],
        out_specs=pl.BlockSpec((None, Ho, Wh, C), lambda n: (n, 0, 0, 0)),
        scratch_shapes=[pltpu.VMEM((R, Wh, C), jnp.float32)],
        compiler_params=pltpu.CompilerParams(dimension_semantics=("parallel",)),
    )(oe, oo)
    return out[:, :, :Ho, :]


def _block_kernel(x_ref, wa_ref, ba_ref, wb_ref, sc_ref, sh_ref, o_ref, sp_ref,
                  *, P, Ho, W2):
    Qa = Ho * P
    acc = jnp.dot(x_ref[0:Qa, :], wa_ref[0], preferred_element_type=jnp.float32)
    for t in range(1, 4):
        off = (t // 2) * P + (t % 2)
        acc = acc + jnp.dot(x_ref[off:off + Qa, :], wa_ref[t],
                            preferred_element_type=jnp.float32)
    ya = jnp.maximum(acc + ba_ref[...], 0.0).astype(jnp.bfloat16)
    ya = ya.reshape(Ho, P, ya.shape[-1])
    sp_ref[...] = jnp.zeros_like(sp_ref)
    sp_ref[1:Ho + 1, 1:Ho + 1, :] = ya[:, 0:Ho, :]
    Qb = Ho * (Ho + 2)
    accb = None
    for ti in range(3):
        for tj in range(3):
            a = sp_ref[ti:ti + Ho, tj:tj + Ho + 2, :].reshape(Qb, sp_ref.shape[-1])
            d = jnp.dot(a, wb_ref[3 * ti + tj], preferred_element_type=jnp.float32)
            accb = d if accb is None else accb + d
    y = jnp.maximum(accb * sc_ref[...] + sh_ref[...], 0.0)
    o_ref[...] = y.astype(o_ref.dtype)


def _block(x, wa, shift_a, wb, scale_b, shift_b):
    B, Hi, _, Ci = x.shape
    Co = wb.shape[-1]
    Ho = Hi // 2
    xp, P = _s2d(x, 1)
    Qa = Ho * P
    Sp = _rup(P + 1 + Qa, 16)
    xp = jnp.pad(xp, ((0, 0), (0, Sp - P * P), (0, 0)))
    wa4 = _pack_w_s2(wa, 3, Ci, Co)
    W2 = Ho + 4
    Qb = Ho * (Ho + 2)
    out = pl.pallas_call(
        functools.partial(_block_kernel, P=P, Ho=Ho, W2=W2),
        out_shape=jax.ShapeDtypeStruct((B, Qb, Co), jnp.bfloat16),
        grid=(B,),
        in_specs=[
            pl.BlockSpec((None, Sp, 4 * Ci), lambda n: (n, 0, 0)),
            pl.BlockSpec((4, 4 * Ci, Co), lambda n: (0, 0, 0)),
            pl.BlockSpec((1, Co), lambda n: (0, 0)),
            pl.BlockSpec((9, Co, Co), lambda n: (0, 0, 0)),
            pl.BlockSpec((1, Co), lambda n: (0, 0)),
            pl.BlockSpec((1, Co), lambda n: (0, 0)),
        ],
        out_specs=pl.BlockSpec((None, Qb, Co), lambda n: (n, 0, 0)),
        scratch_shapes=[pltpu.VMEM((Ho + 2, W2, Co), jnp.bfloat16)],
        compiler_params=pltpu.CompilerParams(dimension_semantics=("parallel",)),
    )(xp, wa4, shift_a, wb, scale_b, shift_b)
    return out.reshape(B, Ho, Ho + 2, Co)[:, :, :Ho, :]


def _gap_kernel(x_ref, w_ref, b_ref, o_ref, *, inv_hw):
    z = jnp.sum(x_ref[...].astype(jnp.float32), axis=1) * inv_hw
    o_ref[...] = jnp.dot(z.astype(jnp.bfloat16), w_ref[...],
                         preferred_element_type=jnp.float32) + b_ref[...]


def _gap_linear(x, w, b):
    B, HW, C = x.shape
    ncls = w.shape[1]
    return pl.pallas_call(
        functools.partial(_gap_kernel, inv_hw=1.0 / HW),
        out_shape=jax.ShapeDtypeStruct((B, ncls), jnp.float32),
        grid=(1,),
        in_specs=[
            pl.BlockSpec((B, HW, C), lambda i: (0, 0, 0)),
            pl.BlockSpec((C, ncls), lambda i: (0, 0)),
            pl.BlockSpec((1, ncls), lambda i: (0, 0)),
        ],
        out_specs=pl.BlockSpec((B, ncls), lambda i: (0, 0)),
    )(x, w, b)


def kernel(x_nchw, norm_mean, norm_std, stem_w, stem_shift, fc_w, fc_b,
           b0_wa, b0_shift_a, b0_wb, b0_scale_b, b0_shift_b,
           b1_wa, b1_shift_a, b1_wb, b1_scale_b, b1_shift_b,
           b2_wa, b2_shift_a, b2_wb, b2_scale_b, b2_shift_b):
    oe, oo = _stem(x_nchw, norm_mean, norm_std, stem_w, stem_shift)
    x = _maxpool(oe, oo)
    for (wa, sa, wb, sc, sh) in (
            (b0_wa, b0_shift_a, b0_wb, b0_scale_b, b0_shift_b),
            (b1_wa, b1_shift_a, b1_wb, b1_scale_b, b1_shift_b),
            (b2_wa, b2_shift_a, b2_wb, b2_scale_b, b2_shift_b)):
        x = _block(x, wa, sa, wb, sc, sh)
    B, H, _, C = x.shape
    return _gap_linear(x.reshape(B, H * H, C), fc_w, fc_b)
